# speedup vs baseline: 1.2000x; 1.2000x over previous
import functools

import jax
import jax.numpy as jnp
from jax import lax
from jax.experimental import pallas as pl
from jax.experimental.pallas import tpu as pltpu


def _ffn_kernel(x_ref, w1_ref, b1_ref, w2_ref, b2_ref, o_ref):
    h = jnp.dot(x_ref[...], w1_ref[...], preferred_element_type=jnp.float32)
    h = h + b1_ref[...]
    h = 0.5 * h * (1.0 + lax.erf(h * 0.7071067811865476))
    out = jnp.dot(h.astype(jnp.bfloat16), w2_ref[...],
                  preferred_element_type=jnp.float32)
    o_ref[...] = out + b2_ref[...]


@functools.partial(jax.jit, static_argnames=("block_rows",))
def kernel(x, w1, b1, w2, b2, *, block_rows=256):
    orig_lead = x.shape[:-1]
    C_in = x.shape[-1]
    H = w1.shape[1]
    C_out = w2.shape[1]
    rows = 1
    for d in orig_lead:
        rows *= d

    x2 = x.reshape(rows, C_in).astype(jnp.bfloat16)
    w1b = w1.astype(jnp.bfloat16)
    w2b = w2.astype(jnp.bfloat16)
    b1r = b1.astype(jnp.float32).reshape(1, H)
    b2r = b2.astype(jnp.float32).reshape(1, C_out)

    bm = min(block_rows, rows)
    n_row = pl.cdiv(rows, bm)

    out2d = pl.pallas_call(
        _ffn_kernel,
        out_shape=jax.ShapeDtypeStruct((rows, C_out), jnp.float32),
        grid=(n_row,),
        in_specs=[
            pl.BlockSpec((bm, C_in), lambda i: (i, 0)),
            pl.BlockSpec((C_in, H), lambda i: (0, 0)),
            pl.BlockSpec((1, H), lambda i: (0, 0)),
            pl.BlockSpec((H, C_out), lambda i: (0, 0)),
            pl.BlockSpec((1, C_out), lambda i: (0, 0)),
        ],
        out_specs=pl.BlockSpec((bm, C_out), lambda i: (i, 0)),
        compiler_params=pltpu.CompilerParams(
            dimension_semantics=("parallel",),
            vmem_limit_bytes=60 << 20,
        ),
    )(x2, w1b, b1r, w2b, b2r)

    return out2d.reshape(*orig_lead, C_out).astype(x.dtype)


# speedup vs baseline: 1.4266x; 1.1889x over previous
import functools

import jax
import jax.numpy as jnp
from jax import lax
from jax.experimental import pallas as pl
from jax.experimental.pallas import tpu as pltpu


def _ffn_kernel(x_ref, w1_ref, b1_ref, w2_ref, b2_ref, o_ref, xb_ref):
    h_idx = pl.program_id(1)

    @pl.when(h_idx == 0)
    def _init():
        xb_ref[...] = x_ref[...].astype(jnp.bfloat16)
        o_ref[...] = jnp.broadcast_to(b2_ref[...], o_ref.shape)

    t = jnp.dot(xb_ref[...], w1_ref[...].astype(jnp.bfloat16),
                preferred_element_type=jnp.float32)
    t = t + b1_ref[...]
    t = 0.5 * t * (1.0 + lax.erf(t * 0.7071067811865476))
    o_ref[...] += jnp.dot(t.astype(jnp.bfloat16),
                          w2_ref[...].astype(jnp.bfloat16),
                          preferred_element_type=jnp.float32)


@functools.partial(jax.jit, static_argnames=("row_blocks", "block_hidden"))
def kernel(x, w1, b1, w2, b2, *, row_blocks=2, block_hidden=512):
    orig_lead = x.shape[:-1]
    C_in = x.shape[-1]
    H = w1.shape[1]
    C_out = w2.shape[1]
    rows = 1
    for d in orig_lead:
        rows *= d

    x2 = x.reshape(rows, C_in)
    b1r = b1.reshape(1, H)
    b2r = b2.reshape(1, C_out)

    bm = rows // row_blocks
    th = min(block_hidden, H)
    n_h = H // th

    out2d = pl.pallas_call(
        _ffn_kernel,
        out_shape=jax.ShapeDtypeStruct((rows, C_out), jnp.float32),
        grid=(row_blocks, n_h),
        in_specs=[
            pl.BlockSpec((bm, C_in), lambda i, h: (i, 0)),
            pl.BlockSpec((C_in, th), lambda i, h: (0, h)),
            pl.BlockSpec((1, th), lambda i, h: (0, h)),
            pl.BlockSpec((th, C_out), lambda i, h: (h, 0)),
            pl.BlockSpec((1, C_out), lambda i, h: (0, 0)),
        ],
        out_specs=pl.BlockSpec((bm, C_out), lambda i, h: (i, 0)),
        scratch_shapes=[pltpu.VMEM((bm, C_in), jnp.bfloat16)],
        compiler_params=pltpu.CompilerParams(
            dimension_semantics=("parallel", "arbitrary"),
            vmem_limit_bytes=64 << 20,
        ),
    )(x2, w1, b1r, w2, b2r)

    return out2d.reshape(*orig_lead, C_out).astype(x.dtype)


# speedup vs baseline: 1.4446x; 1.0126x over previous
import functools

import jax
import jax.numpy as jnp
from jax import lax
from jax.experimental import pallas as pl
from jax.experimental.pallas import tpu as pltpu


def _ffn_kernel(x_ref, w1_ref, b1_ref, w2_ref, b2_ref, o_ref, xb_ref):
    h_idx = pl.program_id(1)

    @pl.when(h_idx == 0)
    def _init():
        xb_ref[...] = x_ref[...].astype(jnp.bfloat16)
        o_ref[...] = jnp.broadcast_to(b2_ref[...], o_ref.shape)

    t = jnp.dot(xb_ref[...], w1_ref[...].astype(jnp.bfloat16),
                preferred_element_type=jnp.float32)
    t = t + b1_ref[...]
    t = 0.5 * t * (1.0 + lax.erf(t * 0.7071067811865476))
    o_ref[...] += jnp.dot(t.astype(jnp.bfloat16),
                          w2_ref[...].astype(jnp.bfloat16),
                          preferred_element_type=jnp.float32)


@functools.partial(jax.jit, static_argnames=("row_blocks", "block_hidden"))
def kernel(x, w1, b1, w2, b2, *, row_blocks=2, block_hidden=1024):
    orig_lead = x.shape[:-1]
    C_in = x.shape[-1]
    H = w1.shape[1]
    C_out = w2.shape[1]
    rows = 1
    for d in orig_lead:
        rows *= d

    x2 = x.reshape(rows, C_in)
    b1r = b1.reshape(1, H)
    b2r = b2.reshape(1, C_out)

    bm = rows // row_blocks
    th = min(block_hidden, H)
    n_h = H // th

    out2d = pl.pallas_call(
        _ffn_kernel,
        out_shape=jax.ShapeDtypeStruct((rows, C_out), jnp.float32),
        grid=(row_blocks, n_h),
        in_specs=[
            pl.BlockSpec((bm, C_in), lambda i, h: (i, 0)),
            pl.BlockSpec((C_in, th), lambda i, h: (0, h)),
            pl.BlockSpec((1, th), lambda i, h: (0, h)),
            pl.BlockSpec((th, C_out), lambda i, h: (h, 0)),
            pl.BlockSpec((1, C_out), lambda i, h: (0, 0)),
        ],
        out_specs=pl.BlockSpec((bm, C_out), lambda i, h: (i, 0)),
        scratch_shapes=[pltpu.VMEM((bm, C_in), jnp.bfloat16)],
        compiler_params=pltpu.CompilerParams(
            dimension_semantics=("parallel", "arbitrary"),
            vmem_limit_bytes=64 << 20,
        ),
    )(x2, w1, b1r, w2, b2r)

    return out2d.reshape(*orig_lead, C_out).astype(x.dtype)
